# baseline (speedup 1.0000x reference)
import jax
import jax.numpy as jnp
from jax import lax
from jax.experimental import pallas as pl
from jax.experimental.pallas import tpu as pltpu
from jax.experimental.pallas import tpu_sc as plsc

M = 1000000
D = 16
B = 16384
NC = 2
NS = 16
NW = NC * NS
LANES = 16
RANGE = 62504
TPAD = NS * RANGE
BPW = B // NW
NVREG = B // LANES


def _body(idx_hbm, val_hbm, out_hbm, t_hbm, idx_v, tbl_v, win_v, rows_v, sem):
    c = lax.axis_index("c")
    s = lax.axis_index("s")
    lo = s * RANGE
    pltpu.sync_copy(idx_hbm, idx_v)

    def step(g, carry):
        for k in range(4):
            v = g * 4 + k
            ids = idx_v[pl.ds(v * LANES, LANES)]
            j = v * LANES + lax.iota(jnp.int32, LANES)
            mask = (ids >= lo) & (ids < lo + RANGE)
            unused_cnt, last = plsc.scan_count(ids, mask=mask)
            keep = mask & last
            loc = jnp.where(keep, ids - lo, 0)
            plsc.store_scatter(tbl_v, [loc], j, mask=keep)
        return carry

    lax.fori_loop(0, NVREG // 4, step, None)
    pltpu.sync_copy(tbl_v, t_hbm.at[pl.ds(lo, RANGE)])
    plsc.subcore_barrier()

    wid = s * NC + c
    base = wid * BPW
    pltpu.async_copy(t_hbm.at[idx_v.at[pl.ds(base, BPW)]], win_v, sem).wait()
    pltpu.async_copy(val_hbm.at[win_v], rows_v, sem).wait()
    pltpu.sync_copy(rows_v, out_hbm.at[pl.ds(base, BPW)])


def kernel(mem, idx, val):
    del mem
    mesh = plsc.VectorSubcoreMesh(core_axis_name="c", subcore_axis_name="s")

    fused = pl.kernel(
        _body,
        out_type=(
            jax.ShapeDtypeStruct((B, D), jnp.float32),
            jax.ShapeDtypeStruct((TPAD,), jnp.int32),
        ),
        mesh=mesh,
        compiler_params=pltpu.CompilerParams(
            needs_layout_passes=False,
            use_tc_tiling_on_sc=False,
        ),
        scratch_types=[
            pltpu.VMEM((B,), jnp.int32),
            pltpu.VMEM((RANGE,), jnp.int32),
            pltpu.VMEM((BPW,), jnp.int32),
            pltpu.VMEM((BPW, D), jnp.float32),
            pltpu.SemaphoreType.DMA,
        ],
    )
    out, _ = fused(idx, val)
    return out
